# Initial kernel scaffold; baseline (speedup 1.0000x reference)
import jax
import jax.numpy as jnp
from jax.experimental import pallas as pl


def kernel(x, base_W, base_b, router_W, lora_A, lora_B):
    raise NotImplementedError("write your pallas kernel here")


# speedup vs baseline: 8.7729x; 8.7729x over previous
import jax
import jax.numpy as jnp
from jax.experimental import pallas as pl
from jax.experimental.pallas import tpu as pltpu

IN_F = 1024
OUT_F = 1024
RANK = 16
NE = 16
SCALING = 2.0
TB = 256


def _routing_weights(logits):
    m = jnp.max(logits, axis=-1, keepdims=True)
    e = jnp.exp(logits - m)
    iota = jax.lax.broadcasted_iota(jnp.int32, e.shape, 1)
    i1 = jnp.min(jnp.where(e == 1.0, iota, NE), axis=-1, keepdims=True)
    oh1 = iota == i1
    em = jnp.where(oh1, -1.0, e)
    m2 = jnp.max(em, axis=-1, keepdims=True)
    i2 = jnp.min(jnp.where(em == m2, iota, NE), axis=-1, keepdims=True)
    sel = oh1 | (iota == i2)
    return jnp.where(sel, e, 0.0) / (1.0 + m2)


def _fused_kernel(x_ref, bwt_ref, bb_ref, rwt_ref, at_ref, bf_ref, out_ref):
    xb = x_ref[...]
    logits = jnp.dot(xb, rwt_ref[...], preferred_element_type=jnp.float32)
    w = _routing_weights(logits)
    h = jnp.dot(xb, at_ref[...], preferred_element_type=jnp.float32)
    er = jax.lax.broadcasted_iota(jnp.int32, (NE, NE * RANK), 0)
    ec = jax.lax.broadcasted_iota(jnp.int32, (NE, NE * RANK), 1)
    expand = (ec // RANK == er).astype(jnp.float32)
    hw = h * jnp.dot(w, expand, preferred_element_type=jnp.float32)
    base = jnp.dot(xb, bwt_ref[...], preferred_element_type=jnp.float32)
    lora = jnp.dot(hw, bf_ref[...], preferred_element_type=jnp.float32)
    out_ref[...] = base + bb_ref[...] + SCALING * lora


def kernel(x, base_W, base_b, router_W, lora_A, lora_B):
    orig_shape = x.shape
    x_flat = x.reshape(-1, IN_F)
    n_tok = x_flat.shape[0]
    grid = (n_tok // TB,)

    bwt = base_W.T
    rwt = router_W.T
    at = lora_A.reshape(NE * RANK, IN_F).T
    bf = lora_B.transpose(0, 2, 1).reshape(NE * RANK, OUT_F)
    bb = base_b.reshape(1, OUT_F)

    out = pl.pallas_call(
        _fused_kernel,
        grid=grid,
        in_specs=[
            pl.BlockSpec((TB, IN_F), lambda i: (i, 0)),
            pl.BlockSpec((IN_F, OUT_F), lambda i: (0, 0)),
            pl.BlockSpec((1, OUT_F), lambda i: (0, 0)),
            pl.BlockSpec((IN_F, NE), lambda i: (0, 0)),
            pl.BlockSpec((IN_F, NE * RANK), lambda i: (0, 0)),
            pl.BlockSpec((NE * RANK, OUT_F), lambda i: (0, 0)),
        ],
        out_specs=pl.BlockSpec((TB, OUT_F), lambda i: (i, 0)),
        out_shape=jax.ShapeDtypeStruct((n_tok, OUT_F), x.dtype),
        compiler_params=pltpu.CompilerParams(
            dimension_semantics=("arbitrary",),
        ),
    )(x_flat, bwt, bb, rwt, at, bf)
    return out.reshape(*orig_shape[:-1], OUT_F)


# speedup vs baseline: 9.4814x; 1.0808x over previous
import jax
import jax.numpy as jnp
from jax.experimental import pallas as pl
from jax.experimental.pallas import tpu as pltpu

IN_F = 1024
OUT_F = 1024
RANK = 16
NE = 16
SCALING = 2.0
TB = 512


def _routing_weights(logits):
    m = jnp.max(logits, axis=-1, keepdims=True)
    e = jnp.exp(logits - m)
    iota = jax.lax.broadcasted_iota(jnp.int32, e.shape, 1)
    i1 = jnp.min(jnp.where(e == 1.0, iota, NE), axis=-1, keepdims=True)
    oh1 = iota == i1
    em = jnp.where(oh1, -1.0, e)
    m2 = jnp.max(em, axis=-1, keepdims=True)
    i2 = jnp.min(jnp.where(em == m2, iota, NE), axis=-1, keepdims=True)
    sel = oh1 | (iota == i2)
    return jnp.where(sel, e, 0.0) / (1.0 + m2)


def _fused_kernel(x_ref, wcat_ref, bb_ref, rwt_ref, bf_ref, out_ref):
    xb = x_ref[...]
    logits = jnp.dot(xb, rwt_ref[...], preferred_element_type=jnp.float32)
    w = _routing_weights(logits)
    xb16 = xb.astype(jnp.bfloat16)
    res = jnp.dot(xb16, wcat_ref[...], preferred_element_type=jnp.float32)
    base = res[:, :OUT_F]
    h = res[:, OUT_F:]
    er = jax.lax.broadcasted_iota(jnp.int32, (NE, NE * RANK), 0)
    ec = jax.lax.broadcasted_iota(jnp.int32, (NE, NE * RANK), 1)
    expand = (ec // RANK == er).astype(jnp.float32)
    hw = (h * jnp.dot(w, expand, preferred_element_type=jnp.float32)).astype(jnp.bfloat16)
    lora = jnp.dot(hw, bf_ref[...], preferred_element_type=jnp.float32)
    out_ref[...] = base + bb_ref[...] + lora


def kernel(x, base_W, base_b, router_W, lora_A, lora_B):
    orig_shape = x.shape
    x_flat = x.reshape(-1, IN_F)
    n_tok = x_flat.shape[0]
    grid = (n_tok // TB,)

    bwt = base_W.T.astype(jnp.bfloat16)
    at = lora_A.reshape(NE * RANK, IN_F).T.astype(jnp.bfloat16)
    wcat = jnp.concatenate([bwt, at], axis=1)
    rwt = router_W.T
    bf = (lora_B.transpose(0, 2, 1).reshape(NE * RANK, OUT_F) * SCALING).astype(jnp.bfloat16)
    bb = base_b.reshape(1, OUT_F)

    out = pl.pallas_call(
        _fused_kernel,
        grid=grid,
        in_specs=[
            pl.BlockSpec((TB, IN_F), lambda i: (i, 0)),
            pl.BlockSpec((IN_F, OUT_F + NE * RANK), lambda i: (0, 0)),
            pl.BlockSpec((1, OUT_F), lambda i: (0, 0)),
            pl.BlockSpec((IN_F, NE), lambda i: (0, 0)),
            pl.BlockSpec((NE * RANK, OUT_F), lambda i: (0, 0)),
        ],
        out_specs=pl.BlockSpec((TB, OUT_F), lambda i: (i, 0)),
        out_shape=jax.ShapeDtypeStruct((n_tok, OUT_F), x.dtype),
        compiler_params=pltpu.CompilerParams(
            dimension_semantics=("arbitrary",),
        ),
    )(x_flat, wcat, bb, rwt, bf)
    return out.reshape(*orig_shape[:-1], OUT_F)


# speedup vs baseline: 9.7953x; 1.0331x over previous
import jax
import jax.numpy as jnp
from jax.experimental import pallas as pl
from jax.experimental.pallas import tpu as pltpu

IN_F = 1024
OUT_F = 1024
RANK = 16
NE = 16
SCALING = 2.0
TB = 512

_DN_T = (((1,), (1,)), ((), ()))


def _routing_weights(logits):
    m = jnp.max(logits, axis=-1, keepdims=True)
    e = jnp.exp(logits - m)
    iota = jax.lax.broadcasted_iota(jnp.int32, e.shape, 1)
    i1 = jnp.min(jnp.where(e == 1.0, iota, NE), axis=-1, keepdims=True)
    oh1 = iota == i1
    em = jnp.where(oh1, -1.0, e)
    m2 = jnp.max(em, axis=-1, keepdims=True)
    i2 = jnp.min(jnp.where(em == m2, iota, NE), axis=-1, keepdims=True)
    sel = oh1 | (iota == i2)
    return jnp.where(sel, e, 0.0) / (1.0 + m2)


def _fused_kernel(x_ref, bw_ref, bb_ref, rw_ref, a_ref, bf_ref, out_ref):
    xb = x_ref[...]
    logits = jax.lax.dot_general(xb, rw_ref[...], _DN_T,
                                 preferred_element_type=jnp.float32)
    w = _routing_weights(logits)
    xb16 = xb.astype(jnp.bfloat16)
    base = jax.lax.dot_general(xb16, bw_ref[...], _DN_T,
                               preferred_element_type=jnp.float32)
    h = jax.lax.dot_general(xb16, a_ref[...], _DN_T,
                            preferred_element_type=jnp.float32)
    er = jax.lax.broadcasted_iota(jnp.int32, (NE, NE * RANK), 0)
    ec = jax.lax.broadcasted_iota(jnp.int32, (NE, NE * RANK), 1)
    expand = (ec // RANK == er).astype(jnp.float32)
    hw = (h * jnp.dot(w, expand,
                      preferred_element_type=jnp.float32)).astype(jnp.bfloat16)
    lora = jnp.dot(hw, bf_ref[...], preferred_element_type=jnp.float32)
    out_ref[...] = base + bb_ref[...] + lora


def kernel(x, base_W, base_b, router_W, lora_A, lora_B):
    orig_shape = x.shape
    x_flat = x.reshape(-1, IN_F)
    n_tok = x_flat.shape[0]
    grid = (n_tok // TB,)

    bw16 = base_W.astype(jnp.bfloat16)
    a16 = lora_A.reshape(NE * RANK, IN_F).astype(jnp.bfloat16)
    bf = (lora_B.transpose(0, 2, 1).reshape(NE * RANK, OUT_F) * SCALING).astype(jnp.bfloat16)
    bb = base_b.reshape(1, OUT_F)

    out = pl.pallas_call(
        _fused_kernel,
        grid=grid,
        in_specs=[
            pl.BlockSpec((TB, IN_F), lambda i: (i, 0)),
            pl.BlockSpec((OUT_F, IN_F), lambda i: (0, 0)),
            pl.BlockSpec((1, OUT_F), lambda i: (0, 0)),
            pl.BlockSpec((NE, IN_F), lambda i: (0, 0)),
            pl.BlockSpec((NE * RANK, IN_F), lambda i: (0, 0)),
            pl.BlockSpec((NE * RANK, OUT_F), lambda i: (0, 0)),
        ],
        out_specs=pl.BlockSpec((TB, OUT_F), lambda i: (i, 0)),
        out_shape=jax.ShapeDtypeStruct((n_tok, OUT_F), x.dtype),
        compiler_params=pltpu.CompilerParams(
            dimension_semantics=("arbitrary",),
        ),
    )(x_flat, bw16, bb, router_W, a16, bf)
    return out.reshape(*orig_shape[:-1], OUT_F)
